# baseline (speedup 1.0000x reference)
import functools

import jax
import jax.numpy as jnp
from jax import lax
from jax.experimental import pallas as pl
from jax.experimental.pallas import tpu as pltpu
from jax.experimental.pallas import tpu_sc as plsc

_NUM_VARS = 26
_CARD = 100000
_EMB = 64
_BATCH = 4096
_NC = 2
_NS = 16
_NW = _NC * _NS
_ROWS = _BATCH // _NW


def _build_kernel():
    mesh = plsc.VectorSubcoreMesh(core_axis_name="c", subcore_axis_name="s")
    out_type = tuple(
        jax.ShapeDtypeStruct((_BATCH, _EMB), jnp.float32)
        for _ in range(_NUM_VARS)
    )

    @functools.partial(
        pl.kernel,
        mesh=mesh,
        out_type=out_type,
        compiler_params=pltpu.CompilerParams(use_tc_tiling_on_sc=False),
        scratch_types=[
            pltpu.VMEM((_NUM_VARS, _ROWS), jnp.int32),
            pltpu.VMEM((_ROWS, _EMB), jnp.float32),
            pltpu.VMEM((_ROWS, _EMB), jnp.float32),
            pltpu.SemaphoreType.DMA,
        ],
    )
    def k(idx_hbm, *rest):
        table_refs = rest[:_NUM_VARS]
        out_refs = rest[_NUM_VARS:2 * _NUM_VARS]
        idx_v, buf0, buf1, sem = rest[2 * _NUM_VARS:]

        wid = lax.axis_index("s") * _NC + lax.axis_index("c")
        base = wid * _ROWS

        pltpu.sync_copy(idx_hbm.at[:, pl.ds(base, _ROWS)], idx_v)

        bufs = (buf0, buf1)
        for t in range(_NUM_VARS):
            buf = bufs[t % 2]
            pltpu.async_copy(table_refs[t].at[idx_v.at[t]], buf, sem).wait()
            pltpu.sync_copy(buf, out_refs[t].at[pl.ds(base, _ROWS)])

    return k


_sc_embed = _build_kernel()


def kernel(x, tables):
    xt = x.T
    return _sc_embed(xt, *tables)


# speedup vs baseline: 1.0105x; 1.0105x over previous
import functools

import jax
import jax.numpy as jnp
from jax import lax
from jax.experimental import pallas as pl
from jax.experimental.pallas import tpu as pltpu
from jax.experimental.pallas import tpu_sc as plsc

_NUM_VARS = 26
_CARD = 100000
_EMB = 64
_BATCH = 4096
_NC = 2
_NS = 16
_NW = _NC * _NS
_ROWS = _BATCH // _NW
_NBUF = 8
_LAG = 6


def _build_kernel():
    mesh = plsc.VectorSubcoreMesh(core_axis_name="c", subcore_axis_name="s")
    out_type = tuple(
        jax.ShapeDtypeStruct((_BATCH, _EMB), jnp.float32)
        for _ in range(_NUM_VARS)
    )

    @functools.partial(
        pl.kernel,
        mesh=mesh,
        out_type=out_type,
        compiler_params=pltpu.CompilerParams(use_tc_tiling_on_sc=False),
        scratch_types=(
            [pltpu.VMEM((_NUM_VARS, _ROWS), jnp.int32)]
            + [pltpu.VMEM((_ROWS, _EMB), jnp.float32) for _ in range(_NBUF)]
            + [pltpu.SemaphoreType.DMA for _ in range(2 * _NBUF)]
        ),
    )
    def k(idx_hbm, *rest):
        table_refs = rest[:_NUM_VARS]
        out_refs = rest[_NUM_VARS:2 * _NUM_VARS]
        scratch = rest[2 * _NUM_VARS:]
        idx_v = scratch[0]
        bufs = scratch[1:1 + _NBUF]
        gsems = scratch[1 + _NBUF:1 + 2 * _NBUF]
        osems = scratch[1 + 2 * _NBUF:1 + 3 * _NBUF]

        wid = lax.axis_index("s") * _NC + lax.axis_index("c")
        base = wid * _ROWS

        pltpu.sync_copy(idx_hbm.at[:, pl.ds(base, _ROWS)], idx_v)

        copies_g = [None] * _NUM_VARS
        copies_o = [None] * _NUM_VARS

        def start_out(u):
            copies_g[u].wait()
            s = u % _NBUF
            copies_o[u] = pltpu.async_copy(
                bufs[s], out_refs[u].at[pl.ds(base, _ROWS)], osems[s])

        for t in range(_NUM_VARS):
            s = t % _NBUF
            if t >= _NBUF:
                copies_o[t - _NBUF].wait()
            copies_g[t] = pltpu.async_copy(
                table_refs[t].at[idx_v.at[t]], bufs[s], gsems[s])
            if t >= _LAG:
                start_out(t - _LAG)
        for u in range(_NUM_VARS - _LAG, _NUM_VARS):
            start_out(u)
        for u in range(_NUM_VARS - _NBUF, _NUM_VARS):
            copies_o[u].wait()

    return k


_sc_embed = _build_kernel()


def kernel(x, tables):
    xt = x.T
    return _sc_embed(xt, *tables)


# speedup vs baseline: 1.1702x; 1.1581x over previous
import functools

import jax
import jax.numpy as jnp
from jax import lax
from jax.experimental import pallas as pl
from jax.experimental.pallas import tpu as pltpu
from jax.experimental.pallas import tpu_sc as plsc

_NUM_VARS = 26
_CARD = 100000
_EMB = 64
_BATCH = 4096
_NC = 2
_NS = 16
_NW = _NC * _NS
_ROWS = _BATCH // _NW
_NBUF = 3


def _build_kernel():
    mesh = plsc.VectorSubcoreMesh(core_axis_name="c", subcore_axis_name="s")
    out_type = tuple(
        jax.ShapeDtypeStruct((_EMB, _BATCH), jnp.float32)
        for _ in range(_NUM_VARS)
    )

    @functools.partial(
        pl.kernel,
        mesh=mesh,
        out_type=out_type,
        compiler_params=pltpu.CompilerParams(use_tc_tiling_on_sc=False),
        scratch_types=(
            [pltpu.VMEM((_NUM_VARS, _ROWS), jnp.int32)]
            + [pltpu.VMEM((_EMB, _ROWS), jnp.float32) for _ in range(_NBUF)]
            + [pltpu.SemaphoreType.DMA for _ in range(2 * _NBUF)]
        ),
    )
    def k(idx_hbm, *rest):
        table_refs = rest[:_NUM_VARS]
        out_refs = rest[_NUM_VARS:2 * _NUM_VARS]
        scratch = rest[2 * _NUM_VARS:]
        idx_v = scratch[0]
        slabs = scratch[1:1 + _NBUF]
        gsems = scratch[1 + _NBUF:1 + 2 * _NBUF]
        osems = scratch[1 + 2 * _NBUF:1 + 3 * _NBUF]

        wid = lax.axis_index("s") * _NC + lax.axis_index("c")
        base = wid * _ROWS

        pltpu.sync_copy(idx_hbm.at[:, pl.ds(base, _ROWS)], idx_v)

        def fire_gathers(t):
            s = t % _NBUF

            @pl.loop(0, _EMB)
            def _(d):
                pltpu.async_copy(
                    table_refs[t].at[d].at[idx_v.at[t]],
                    slabs[s].at[d],
                    gsems[s],
                )

        def drain_and_store(t):
            s = t % _NBUF
            pltpu.make_async_copy(
                out_refs[t].at[:, pl.ds(0, _ROWS)], slabs[s], gsems[s]
            ).wait()
            return pltpu.async_copy(
                slabs[s], out_refs[t].at[:, pl.ds(base, _ROWS)], osems[s])

        out_copies = [None] * _NUM_VARS
        for t in range(_NUM_VARS):
            if t >= _NBUF:
                out_copies[t - _NBUF].wait()
            fire_gathers(t)
            if t >= 1:
                out_copies[t - 1] = drain_and_store(t - 1)
        out_copies[_NUM_VARS - 1] = drain_and_store(_NUM_VARS - 1)
        for t in range(_NUM_VARS - _NBUF, _NUM_VARS):
            out_copies[t].wait()

    return k


_sc_embed = _build_kernel()


def kernel(x, tables):
    xt = x.T
    tabts = tuple(t.T for t in tables)
    outs = _sc_embed(xt, *tabts)
    return tuple(o.T for o in outs)
